# baseline (speedup 1.0000x reference)
import functools

import jax
import jax.numpy as jnp
from jax import lax
from jax.experimental import pallas as pl
from jax.experimental.pallas import tpu as pltpu
from jax.experimental.pallas import tpu_sc as plsc

_NC = 2
_NS = 16
_NW = _NC * _NS
_LANES = 16
_CH = 128
_NBUF = 2
_EB = 2048
_NB = 2000


def _cdiv(a, b):
    return (a + b - 1) // b


def _dot(a, b):
    return jnp.dot(a, b, preferred_element_type=jnp.float32)


def _y_body(x_ref, wt_ref, y_ref):
    y_ref[...] = _dot(x_ref[...], wt_ref[...])


def _z0_body(ef_ref, we_ref, be_ref, wb_ref, bm_ref, z_ref):
    e = jnp.tanh(_dot(ef_ref[...], we_ref[...]) + be_ref[...])
    z_ref[...] = _dot(e, wb_ref[...]) + bm_ref[...]


def _z1_body(m_ref, wb_ref, bm_ref, z_ref):
    z_ref[...] = _dot(m_ref[...], wb_ref[...]) + bm_ref[...]


def _upd_body(x_ref, p_ref, wx_ref, wa_ref, bu_ref, wt_ref, x1_ref, y1_ref):
    aggr = p_ref[0] + p_ref[1]
    x1 = jnp.tanh(_dot(x_ref[...], wx_ref[...]) + _dot(aggr, wa_ref[...])
                  + bu_ref[...])
    x1_ref[...] = x1
    y1_ref[...] = _dot(x1, wt_ref[...])


def _fin_body(x_ref, p_ref, wx_ref, wa_ref, bu_ref, wd1_ref, bd1_ref,
              wd2_ref, bd2_ref, o_ref):
    aggr = p_ref[0] + p_ref[1]
    x2 = jnp.tanh(_dot(x_ref[...], wx_ref[...]) + _dot(aggr, wa_ref[...])
                  + bu_ref[...])
    h = jnp.tanh(_dot(x2, wd1_ref[...]) + bd1_ref[...])
    o_ref[...] = _dot(h, wd2_ref[...]) + bd2_ref[...]


def _full_spec(shape):
    return pl.BlockSpec(shape, lambda i: (0,) * len(shape))


def _node_linear(x, wt):
    n, d = x.shape
    grid = (_cdiv(n, _NB),)
    return pl.pallas_call(
        _y_body,
        grid=grid,
        in_specs=[pl.BlockSpec((_NB, d), lambda i: (i, 0)),
                  _full_spec(wt.shape)],
        out_specs=pl.BlockSpec((_NB, d), lambda i: (i, 0)),
        out_shape=jax.ShapeDtypeStruct((n, d), jnp.float32),
    )(x, wt)


def _edge_z0(ef_p, we, be, wb, bm):
    e_pad, fe = ef_p.shape
    d = wb.shape[1]
    grid = (e_pad // _EB,)
    return pl.pallas_call(
        _z0_body,
        grid=grid,
        in_specs=[pl.BlockSpec((_EB, fe), lambda i: (i, 0)),
                  _full_spec(we.shape), _full_spec(be.shape),
                  _full_spec(wb.shape), _full_spec(bm.shape)],
        out_specs=pl.BlockSpec((_EB, d), lambda i: (i, 0)),
        out_shape=jax.ShapeDtypeStruct((e_pad, d), jnp.float32),
    )(ef_p, we, be, wb, bm)


def _edge_z1(m0, wb, bm):
    e_pad, d = m0.shape
    grid = (e_pad // _EB,)
    return pl.pallas_call(
        _z1_body,
        grid=grid,
        in_specs=[pl.BlockSpec((_EB, d), lambda i: (i, 0)),
                  _full_spec(wb.shape), _full_spec(bm.shape)],
        out_specs=pl.BlockSpec((_EB, d), lambda i: (i, 0)),
        out_shape=jax.ShapeDtypeStruct((e_pad, d), jnp.float32),
    )(m0, wb, bm)


def _update(x, part, wx, wa, bu, wt):
    n, d = x.shape
    grid = (_cdiv(n, _NB),)
    return pl.pallas_call(
        _upd_body,
        grid=grid,
        in_specs=[pl.BlockSpec((_NB, d), lambda i: (i, 0)),
                  pl.BlockSpec((_NC, _NB, d), lambda i: (0, i, 0)),
                  _full_spec(wx.shape), _full_spec(wa.shape),
                  _full_spec(bu.shape), _full_spec(wt.shape)],
        out_specs=[pl.BlockSpec((_NB, d), lambda i: (i, 0)),
                   pl.BlockSpec((_NB, d), lambda i: (i, 0))],
        out_shape=[jax.ShapeDtypeStruct((n, d), jnp.float32),
                   jax.ShapeDtypeStruct((n, d), jnp.float32)],
    )(x, part, wx, wa, bu, wt)


def _final(x, part, wx, wa, bu, wd1, bd1, wd2, bd2):
    n, d = x.shape
    grid = (_cdiv(n, _NB),)
    return pl.pallas_call(
        _fin_body,
        grid=grid,
        in_specs=[pl.BlockSpec((_NB, d), lambda i: (i, 0)),
                  pl.BlockSpec((_NC, _NB, d), lambda i: (0, i, 0)),
                  _full_spec(wx.shape), _full_spec(wa.shape),
                  _full_spec(bu.shape), _full_spec(wd1.shape),
                  _full_spec(bd1.shape), _full_spec(wd2.shape),
                  _full_spec(bd2.shape)],
        out_specs=pl.BlockSpec((_NB, 1), lambda i: (i, 0)),
        out_shape=jax.ShapeDtypeStruct((n, 1), jnp.float32),
    )(x, part, wx, wa, bu, wd1, bd1, wd2, bd2)


def _tanh_inplace(gv, d):
    nvec = d // _LANES

    def row(i, _):
        for j in range(nvec):
            sl = pl.ds(j * _LANES, _LANES)
            u = gv[i, sl]
            uc = jnp.minimum(jnp.maximum(u, -15.0), 15.0)
            t = jnp.exp(uc * 2.0)
            gv[i, sl] = 1.0 - 2.0 / (1.0 + t)
        return 0

    lax.fori_loop(0, _CH, row, 0)


def _sc_edge_body(write_m, n_pad, nch0, nch1, d,
                  y_hbm, z_hbm, si_hbm, di_hbm, zero_hbm, m_hbm, part_hbm,
                  aggr_sh, sidx, didx, gv0, gv1, sz0, sz1, sg0, sg1):
    cid = lax.axis_index("c")
    sid = lax.axis_index("s")
    rpt = n_pad // _NS
    r0 = pl.multiple_of(sid * rpt, 8)

    pltpu.sync_copy(zero_hbm.at[pl.ds(r0, rpt)], aggr_sh.at[pl.ds(r0, rpt)])
    plsc.subcore_barrier()

    gvs = (gv0, gv1)
    semz = (sz0, sz1)
    semg = (sg0, sg1)

    def fetch(g, b):
        pltpu.sync_copy(si_hbm.at[g], sidx.at[b])
        pltpu.sync_copy(di_hbm.at[g], didx.at[b])
        e0 = pl.multiple_of(g * _CH, _CH)
        return pltpu.async_copy(z_hbm.at[pl.ds(e0, _CH)], gvs[b], semz[b])

    def gather(g, b):
        return pltpu.async_copy(y_hbm.at[sidx.at[b]], gvs[b], semg[b],
                                add=True)

    def consume(g, b):
        _tanh_inplace(gvs[b], d)
        e0 = pl.multiple_of(g * _CH, _CH)
        if write_m:
            pltpu.sync_copy(gvs[b], m_hbm.at[pl.ds(e0, _CH)])
        pltpu.sync_copy(gvs[b], aggr_sh.at[didx.at[b]], add=True)

    base = jnp.where(cid == 0, sid * nch0, _NS * nch0 + sid * nch1)
    pairs = jnp.where(cid == 0, nch0 // 2, nch1 // 2)

    def pair(c, _):
        g0 = base + 2 * c
        g1 = g0 + 1
        f0 = fetch(g0, 0)
        f1 = fetch(g1, 1)
        f0.wait()
        cp0 = gather(g0, 0)
        f1.wait()
        cp1 = gather(g1, 1)
        cp0.wait()
        consume(g0, 0)
        cp1.wait()
        consume(g1, 1)
        return 0

    lax.fori_loop(0, pairs, pair, 0, unroll=False)

    plsc.subcore_barrier()
    pltpu.sync_copy(aggr_sh.at[pl.ds(r0, rpt)],
                    part_hbm.at[cid, pl.ds(r0, rpt)])


_FRAC0 = 0.4


def _make_sc_layer(write_m, n_pad, e_pad, d):
    g_tot = e_pad // _CH
    nch0 = 2 * int(round(_FRAC0 * g_tot / _NS / 2))
    nch1 = (g_tot - _NS * nch0) // _NS
    mesh = plsc.VectorSubcoreMesh(core_axis_name="c", subcore_axis_name="s",
                                  num_cores=_NC, num_subcores=_NS)
    part_t = jax.ShapeDtypeStruct((_NC, n_pad, d), jnp.float32)
    if write_m:
        out_type = (jax.ShapeDtypeStruct((e_pad, d), jnp.float32), part_t)
    else:
        out_type = part_t
    scratch = [
        pltpu.VMEM_SHARED((n_pad, d), jnp.float32),
        pltpu.VMEM((_NBUF, 128), jnp.int32),
        pltpu.VMEM((_NBUF, 128), jnp.int32),
        pltpu.VMEM((_CH, d), jnp.float32),
        pltpu.VMEM((_CH, d), jnp.float32),
        pltpu.SemaphoreType.DMA,
        pltpu.SemaphoreType.DMA,
        pltpu.SemaphoreType.DMA,
        pltpu.SemaphoreType.DMA,
    ]

    if write_m:
        def body(y, z, si, di, zero, m, part, *s):
            _sc_edge_body(True, n_pad, nch0, nch1, d, y, z, si, di, zero, m,
                          part, *s)
    else:
        def body(y, z, si, di, zero, part, *s):
            _sc_edge_body(False, n_pad, nch0, nch1, d, y, z, si, di, zero,
                          None, part, *s)

    return pl.kernel(body, out_type=out_type, mesh=mesh,
                     scratch_types=scratch)


def kernel(node_feature, edge_index, edge_feature, batch,
           W_emb, b_emb,
           W_msg0, b_msg0, W_upd0, b_upd0,
           W_msg1, b_msg1, W_upd1, b_upd1,
           W_d1, b_d1, W_d2, b_d2):
    n, d = node_feature.shape
    e = edge_index.shape[1]
    fe = edge_feature.shape[1]

    e_pad = _cdiv(e, _NW * _CH * 2) * (_NW * _CH * 2)
    n_pad = _cdiv(n + 1, _NS * 8) * (_NS * 8)
    g_tot = e_pad // _CH

    src = edge_index[0].astype(jnp.int32)
    dst = edge_index[1].astype(jnp.int32)
    src_p = jnp.concatenate(
        [src, jnp.zeros((e_pad - e,), jnp.int32)]).reshape(g_tot, 128)
    dst_p = jnp.concatenate(
        [dst, jnp.full((e_pad - e,), n, jnp.int32)]).reshape(g_tot, 128)
    ef_p = jnp.zeros((e_pad, fe), jnp.float32).at[:e].set(edge_feature)
    zeros_hbm = jnp.zeros((n_pad, d), jnp.float32)

    be = b_emb.reshape(1, d)
    bm0 = b_msg0.reshape(1, d)
    bm1 = b_msg1.reshape(1, d)
    bu0 = b_upd0.reshape(1, d)
    bu1 = b_upd1.reshape(1, d)
    bd1 = b_d1.reshape(1, -1)
    bd2 = b_d2.reshape(1, -1)

    sc_l0 = _make_sc_layer(True, n_pad, e_pad, d)
    sc_l1 = _make_sc_layer(False, n_pad, e_pad, d)

    y0 = _node_linear(node_feature, W_msg0[:d])
    z0 = _edge_z0(ef_p, W_emb, be, W_msg0[d:], bm0)
    m0, part0 = sc_l0(y0, z0, src_p, dst_p, zeros_hbm)
    x1, y1 = _update(node_feature, part0, W_upd0[:d], W_upd0[d:], bu0,
                     W_msg1[:d])

    z1 = _edge_z1(m0, W_msg1[d:], bm1)
    part1 = sc_l1(y1, z1, src_p, dst_p, zeros_hbm)

    return _final(x1, part1, W_upd1[:d], W_upd1[d:], bu1,
                  W_d1, bd1, W_d2, bd2)


# speedup vs baseline: 1.0194x; 1.0194x over previous
import functools

import jax
import jax.numpy as jnp
from jax import lax
from jax.experimental import pallas as pl
from jax.experimental.pallas import tpu as pltpu
from jax.experimental.pallas import tpu_sc as plsc

_NC = 2
_NS = 16
_NW = _NC * _NS
_LANES = 16
_CH = 128
_IB = 4
_EB = 2048
_NB = 2000
_FRAC0 = 0.6


def _cdiv(a, b):
    return (a + b - 1) // b


def _dot(a, b):
    return jnp.dot(a, b, preferred_element_type=jnp.float32)


def _y_body(x_ref, wt_ref, y_ref):
    y_ref[...] = _dot(x_ref[...], wt_ref[...])


def _z0_body(ef_ref, we_ref, be_ref, wb_ref, bm_ref, z_ref):
    e = jnp.tanh(_dot(ef_ref[...], we_ref[...]) + be_ref[...])
    z_ref[...] = _dot(e, wb_ref[...]) + bm_ref[...]


def _z1_body(m_ref, wb_ref, bm_ref, z_ref):
    z_ref[...] = _dot(m_ref[...], wb_ref[...]) + bm_ref[...]


def _upd_body(x_ref, p_ref, wx_ref, wa_ref, bu_ref, wt_ref, x1_ref, y1_ref):
    aggr = p_ref[0] + p_ref[1]
    x1 = jnp.tanh(_dot(x_ref[...], wx_ref[...]) + _dot(aggr, wa_ref[...])
                  + bu_ref[...])
    x1_ref[...] = x1
    y1_ref[...] = _dot(x1, wt_ref[...])


def _fin_body(x_ref, p_ref, wx_ref, wa_ref, bu_ref, wd1_ref, bd1_ref,
              wd2_ref, bd2_ref, o_ref):
    aggr = p_ref[0] + p_ref[1]
    x2 = jnp.tanh(_dot(x_ref[...], wx_ref[...]) + _dot(aggr, wa_ref[...])
                  + bu_ref[...])
    h = jnp.tanh(_dot(x2, wd1_ref[...]) + bd1_ref[...])
    o_ref[...] = _dot(h, wd2_ref[...]) + bd2_ref[...]


def _full_spec(shape):
    return pl.BlockSpec(shape, lambda i: (0,) * len(shape))


def _node_linear(x, wt):
    n, d = x.shape
    grid = (_cdiv(n, _NB),)
    return pl.pallas_call(
        _y_body,
        grid=grid,
        in_specs=[pl.BlockSpec((_NB, d), lambda i: (i, 0)),
                  _full_spec(wt.shape)],
        out_specs=pl.BlockSpec((_NB, d), lambda i: (i, 0)),
        out_shape=jax.ShapeDtypeStruct((n, d), jnp.float32),
    )(x, wt)


def _edge_z0(ef_p, we, be, wb, bm):
    e_pad, fe = ef_p.shape
    d = wb.shape[1]
    grid = (e_pad // _EB,)
    return pl.pallas_call(
        _z0_body,
        grid=grid,
        in_specs=[pl.BlockSpec((_EB, fe), lambda i: (i, 0)),
                  _full_spec(we.shape), _full_spec(be.shape),
                  _full_spec(wb.shape), _full_spec(bm.shape)],
        out_specs=pl.BlockSpec((_EB, d), lambda i: (i, 0)),
        out_shape=jax.ShapeDtypeStruct((e_pad, d), jnp.float32),
    )(ef_p, we, be, wb, bm)


def _edge_z1(m0, wb, bm):
    e_pad, d = m0.shape
    grid = (e_pad // _EB,)
    return pl.pallas_call(
        _z1_body,
        grid=grid,
        in_specs=[pl.BlockSpec((_EB, d), lambda i: (i, 0)),
                  _full_spec(wb.shape), _full_spec(bm.shape)],
        out_specs=pl.BlockSpec((_EB, d), lambda i: (i, 0)),
        out_shape=jax.ShapeDtypeStruct((e_pad, d), jnp.float32),
    )(m0, wb, bm)


def _update(x, part, wx, wa, bu, wt):
    n, d = x.shape
    grid = (_cdiv(n, _NB),)
    return pl.pallas_call(
        _upd_body,
        grid=grid,
        in_specs=[pl.BlockSpec((_NB, d), lambda i: (i, 0)),
                  pl.BlockSpec((_NC, _NB, d), lambda i: (0, i, 0)),
                  _full_spec(wx.shape), _full_spec(wa.shape),
                  _full_spec(bu.shape), _full_spec(wt.shape)],
        out_specs=[pl.BlockSpec((_NB, d), lambda i: (i, 0)),
                   pl.BlockSpec((_NB, d), lambda i: (i, 0))],
        out_shape=[jax.ShapeDtypeStruct((n, d), jnp.float32),
                   jax.ShapeDtypeStruct((n, d), jnp.float32)],
    )(x, part, wx, wa, bu, wt)


def _final(x, part, wx, wa, bu, wd1, bd1, wd2, bd2):
    n, d = x.shape
    grid = (_cdiv(n, _NB),)
    return pl.pallas_call(
        _fin_body,
        grid=grid,
        in_specs=[pl.BlockSpec((_NB, d), lambda i: (i, 0)),
                  pl.BlockSpec((_NC, _NB, d), lambda i: (0, i, 0)),
                  _full_spec(wx.shape), _full_spec(wa.shape),
                  _full_spec(bu.shape), _full_spec(wd1.shape),
                  _full_spec(bd1.shape), _full_spec(wd2.shape),
                  _full_spec(bd2.shape)],
        out_specs=pl.BlockSpec((_NB, 1), lambda i: (i, 0)),
        out_shape=jax.ShapeDtypeStruct((n, 1), jnp.float32),
    )(x, part, wx, wa, bu, wd1, bd1, wd2, bd2)


def _tanh_inplace(gv, d):
    nvec = d // _LANES

    def row(i, _):
        for j in range(nvec):
            sl = pl.ds(j * _LANES, _LANES)
            u = gv[i, sl]
            uc = jnp.minimum(jnp.maximum(u, -15.0), 15.0)
            t = jnp.exp(uc * 2.0)
            gv[i, sl] = 1.0 - 2.0 / (1.0 + t)
        return 0

    lax.fori_loop(0, _CH, row, 0)


def _sc_edge_body(write_m, n_pad, nch0, nch1, d,
                  y_hbm, z_hbm, si_hbm, di_hbm, zero_hbm, m_hbm, part_hbm,
                  aggr_sh, sidx, didx, gv0, gv1, sz0, sz1, sg0, sg1,
                  sw0, sw1):
    cid = lax.axis_index("c")
    sid = lax.axis_index("s")
    rpt = n_pad // _NS
    r0 = pl.multiple_of(sid * rpt, 8)

    pltpu.sync_copy(zero_hbm.at[pl.ds(r0, rpt)], aggr_sh.at[pl.ds(r0, rpt)])
    plsc.subcore_barrier()

    base = jnp.where(cid == 0, sid * nch0, _NS * nch0 + sid * nch1)
    nblk = jnp.where(cid == 0, nch0 // _IB, nch1 // _IB)

    def finish(g, s, gv, sw):
        e0 = pl.multiple_of(g * _CH, _CH)
        _tanh_inplace(gv, d)
        mw = None
        if write_m:
            mw = pltpu.async_copy(gv, m_hbm.at[pl.ds(e0, _CH)], sw)
        pltpu.sync_copy(gv, aggr_sh.at[didx.at[s]], add=True)
        return mw

    def pair(p, blk_g0):
        g0 = blk_g0 + 2 * p
        g1 = g0 + 1
        s0 = 2 * p
        s1 = s0 + 1
        e0 = pl.multiple_of(g0 * _CH, _CH)
        e1 = pl.multiple_of(g1 * _CH, _CH)
        fz0 = pltpu.async_copy(z_hbm.at[pl.ds(e0, _CH)], gv0, sz0)
        fz1 = pltpu.async_copy(z_hbm.at[pl.ds(e1, _CH)], gv1, sz1)
        fz0.wait()
        cp0 = pltpu.async_copy(y_hbm.at[sidx.at[s0]], gv0, sg0, add=True)
        fz1.wait()
        cp1 = pltpu.async_copy(y_hbm.at[sidx.at[s1]], gv1, sg1, add=True)
        cp0.wait()
        mw0 = finish(g0, s0, gv0, sw0)
        cp1.wait()
        mw1 = finish(g1, s1, gv1, sw1)
        if mw0 is not None:
            mw0.wait()
            mw1.wait()
        return blk_g0

    def block(blk, _):
        bi = base // _IB + blk
        bg = pl.multiple_of(bi * _IB, _IB)
        pltpu.sync_copy(si_hbm.at[bi], sidx)
        pltpu.sync_copy(di_hbm.at[bi], didx)
        lax.fori_loop(0, _IB // 2, pair, bg, unroll=False)
        return 0

    lax.fori_loop(0, nblk, block, 0, unroll=False)

    plsc.subcore_barrier()
    pltpu.sync_copy(aggr_sh.at[pl.ds(r0, rpt)],
                    part_hbm.at[cid, pl.ds(r0, rpt)])


def _make_sc_layer(write_m, n_pad, e_pad, d):
    g_tot = e_pad // _CH
    nch0 = _IB * int(round(_FRAC0 * g_tot / _NS / _IB))
    nch1 = (g_tot - _NS * nch0) // _NS
    mesh = plsc.VectorSubcoreMesh(core_axis_name="c", subcore_axis_name="s",
                                  num_cores=_NC, num_subcores=_NS)
    part_t = jax.ShapeDtypeStruct((_NC, n_pad, d), jnp.float32)
    if write_m:
        out_type = (jax.ShapeDtypeStruct((e_pad, d), jnp.float32), part_t)
    else:
        out_type = part_t
    scratch = [
        pltpu.VMEM_SHARED((n_pad, d), jnp.float32),
        pltpu.VMEM((_IB, _CH), jnp.int32),
        pltpu.VMEM((_IB, _CH), jnp.int32),
        pltpu.VMEM((_CH, d), jnp.float32),
        pltpu.VMEM((_CH, d), jnp.float32),
        pltpu.SemaphoreType.DMA,
        pltpu.SemaphoreType.DMA,
        pltpu.SemaphoreType.DMA,
        pltpu.SemaphoreType.DMA,
        pltpu.SemaphoreType.DMA,
        pltpu.SemaphoreType.DMA,
    ]

    if write_m:
        def body(y, z, si, di, zero, m, part, *s):
            _sc_edge_body(True, n_pad, nch0, nch1, d, y, z, si, di, zero, m,
                          part, *s)
    else:
        def body(y, z, si, di, zero, part, *s):
            _sc_edge_body(False, n_pad, nch0, nch1, d, y, z, si, di, zero,
                          None, part, *s)

    return pl.kernel(body, out_type=out_type, mesh=mesh,
                     scratch_types=scratch)


def kernel(node_feature, edge_index, edge_feature, batch,
           W_emb, b_emb,
           W_msg0, b_msg0, W_upd0, b_upd0,
           W_msg1, b_msg1, W_upd1, b_upd1,
           W_d1, b_d1, W_d2, b_d2):
    n, d = node_feature.shape
    e = edge_index.shape[1]
    fe = edge_feature.shape[1]

    e_pad = _cdiv(e, _NW * _CH * _IB) * (_NW * _CH * _IB)
    n_pad = _cdiv(n + 1, _NS * 8) * (_NS * 8)
    g_tot = e_pad // _CH

    src = edge_index[0].astype(jnp.int32)
    dst = edge_index[1].astype(jnp.int32)
    src_p = jnp.concatenate(
        [src, jnp.zeros((e_pad - e,), jnp.int32)]
    ).reshape(g_tot // _IB, _IB, _CH)
    dst_p = jnp.concatenate(
        [dst, jnp.full((e_pad - e,), n, jnp.int32)]
    ).reshape(g_tot // _IB, _IB, _CH)
    ef_p = jnp.zeros((e_pad, fe), jnp.float32).at[:e].set(edge_feature)
    zeros_hbm = jnp.zeros((n_pad, d), jnp.float32)

    be = b_emb.reshape(1, d)
    bm0 = b_msg0.reshape(1, d)
    bm1 = b_msg1.reshape(1, d)
    bu0 = b_upd0.reshape(1, d)
    bu1 = b_upd1.reshape(1, d)
    bd1 = b_d1.reshape(1, -1)
    bd2 = b_d2.reshape(1, -1)

    sc_l0 = _make_sc_layer(True, n_pad, e_pad, d)
    sc_l1 = _make_sc_layer(False, n_pad, e_pad, d)

    y0 = _node_linear(node_feature, W_msg0[:d])
    z0 = _edge_z0(ef_p, W_emb, be, W_msg0[d:], bm0)
    m0, part0 = sc_l0(y0, z0, src_p, dst_p, zeros_hbm)
    x1, y1 = _update(node_feature, part0, W_upd0[:d], W_upd0[d:], bu0,
                     W_msg1[:d])

    z1 = _edge_z1(m0, W_msg1[d:], bm1)
    part1 = sc_l1(y1, z1, src_p, dst_p, zeros_hbm)

    return _final(x1, part1, W_upd1[:d], W_upd1[d:], bu1,
                  W_d1, bd1, W_d2, bd2)


# speedup vs baseline: 1.1466x; 1.1248x over previous
import functools

import jax
import jax.numpy as jnp
from jax import lax
from jax.experimental import pallas as pl
from jax.experimental.pallas import tpu as pltpu
from jax.experimental.pallas import tpu_sc as plsc

_NC = 2
_NS = 16
_NW = _NC * _NS
_LANES = 16
_CH = 128
_IB = 4
_EB = 2048
_NB = 2000
_FRAC0 = 0.6


def _cdiv(a, b):
    return (a + b - 1) // b


def _dot(a, b):
    return jnp.dot(a, b, preferred_element_type=jnp.float32)


def _y_body(x_ref, wt_ref, y_ref):
    y_ref[...] = _dot(x_ref[...], wt_ref[...])


def _z0_body(ef_ref, we_ref, be_ref, wb_ref, bm_ref, z_ref):
    e = jnp.tanh(_dot(ef_ref[...], we_ref[...]) + be_ref[...])
    z_ref[...] = _dot(e, wb_ref[...]) + bm_ref[...]


def _z1_body(m_ref, wb_ref, bm_ref, z_ref):
    z_ref[...] = _dot(m_ref[...], wb_ref[...]) + bm_ref[...]


def _upd_body(x_ref, p_ref, wx_ref, wa_ref, bu_ref, wt_ref, x1_ref, y1_ref):
    aggr = p_ref[0] + p_ref[1]
    x1 = jnp.tanh(_dot(x_ref[...], wx_ref[...]) + _dot(aggr, wa_ref[...])
                  + bu_ref[...])
    x1_ref[...] = x1
    y1_ref[...] = _dot(x1, wt_ref[...])


def _fin_body(x_ref, p_ref, wx_ref, wa_ref, bu_ref, wd1_ref, bd1_ref,
              wd2_ref, bd2_ref, o_ref):
    aggr = p_ref[0] + p_ref[1]
    x2 = jnp.tanh(_dot(x_ref[...], wx_ref[...]) + _dot(aggr, wa_ref[...])
                  + bu_ref[...])
    h = jnp.tanh(_dot(x2, wd1_ref[...]) + bd1_ref[...])
    o_ref[...] = _dot(h, wd2_ref[...]) + bd2_ref[...]


def _full_spec(shape):
    return pl.BlockSpec(shape, lambda i: (0,) * len(shape))


def _node_linear(x, wt):
    n, d = x.shape
    grid = (_cdiv(n, _NB),)
    return pl.pallas_call(
        _y_body,
        grid=grid,
        in_specs=[pl.BlockSpec((_NB, d), lambda i: (i, 0)),
                  _full_spec(wt.shape)],
        out_specs=pl.BlockSpec((_NB, d), lambda i: (i, 0)),
        out_shape=jax.ShapeDtypeStruct((n, d), jnp.float32),
    )(x, wt)


def _edge_z0(ef_p, we, be, wb, bm):
    e_pad, fe = ef_p.shape
    d = wb.shape[1]
    grid = (e_pad // _EB,)
    return pl.pallas_call(
        _z0_body,
        grid=grid,
        in_specs=[pl.BlockSpec((_EB, fe), lambda i: (i, 0)),
                  _full_spec(we.shape), _full_spec(be.shape),
                  _full_spec(wb.shape), _full_spec(bm.shape)],
        out_specs=pl.BlockSpec((_EB, d), lambda i: (i, 0)),
        out_shape=jax.ShapeDtypeStruct((e_pad, d), jnp.float32),
    )(ef_p, we, be, wb, bm)


def _edge_z1(m0, wb, bm):
    e_pad, d = m0.shape
    grid = (e_pad // _EB,)
    return pl.pallas_call(
        _z1_body,
        grid=grid,
        in_specs=[pl.BlockSpec((_EB, d), lambda i: (i, 0)),
                  _full_spec(wb.shape), _full_spec(bm.shape)],
        out_specs=pl.BlockSpec((_EB, d), lambda i: (i, 0)),
        out_shape=jax.ShapeDtypeStruct((e_pad, d), jnp.float32),
    )(m0, wb, bm)


def _update(x, part, wx, wa, bu, wt):
    n, d = x.shape
    grid = (_cdiv(n, _NB),)
    return pl.pallas_call(
        _upd_body,
        grid=grid,
        in_specs=[pl.BlockSpec((_NB, d), lambda i: (i, 0)),
                  pl.BlockSpec((_NC, _NB, d), lambda i: (0, i, 0)),
                  _full_spec(wx.shape), _full_spec(wa.shape),
                  _full_spec(bu.shape), _full_spec(wt.shape)],
        out_specs=[pl.BlockSpec((_NB, d), lambda i: (i, 0)),
                   pl.BlockSpec((_NB, d), lambda i: (i, 0))],
        out_shape=[jax.ShapeDtypeStruct((n, d), jnp.float32),
                   jax.ShapeDtypeStruct((n, d), jnp.float32)],
    )(x, part, wx, wa, bu, wt)


def _final(x, part, wx, wa, bu, wd1, bd1, wd2, bd2):
    n, d = x.shape
    grid = (_cdiv(n, _NB),)
    return pl.pallas_call(
        _fin_body,
        grid=grid,
        in_specs=[pl.BlockSpec((_NB, d), lambda i: (i, 0)),
                  pl.BlockSpec((_NC, _NB, d), lambda i: (0, i, 0)),
                  _full_spec(wx.shape), _full_spec(wa.shape),
                  _full_spec(bu.shape), _full_spec(wd1.shape),
                  _full_spec(bd1.shape), _full_spec(wd2.shape),
                  _full_spec(bd2.shape)],
        out_specs=pl.BlockSpec((_NB, 1), lambda i: (i, 0)),
        out_shape=jax.ShapeDtypeStruct((n, 1), jnp.float32),
    )(x, part, wx, wa, bu, wd1, bd1, wd2, bd2)


def _tanh_inplace(gv, d):
    nvec = d // _LANES

    def row(i, _):
        for j in range(nvec):
            sl = pl.ds(j * _LANES, _LANES)
            u = gv[i, sl]
            uc = jnp.minimum(jnp.maximum(u, -15.0), 15.0)
            t = jnp.exp(uc * 2.0)
            gv[i, sl] = 1.0 - 2.0 / (1.0 + t)
        return 0

    lax.fori_loop(0, _CH, row, 0)


def _sc_edge_body(write_m, n_pad, nch0, nch1, d,
                  y_hbm, z_hbm, si_hbm, di_hbm, zero_hbm, m_hbm, part_hbm,
                  aggr_sh, sidx, didx, gv0, gv1, sz0, sz1, sg0, sg1,
                  sw0, sw1):
    cid = lax.axis_index("c")
    sid = lax.axis_index("s")
    rpt = n_pad // _NS
    r0 = pl.multiple_of(sid * rpt, 8)

    pltpu.sync_copy(zero_hbm.at[pl.ds(r0, rpt)], aggr_sh.at[pl.ds(r0, rpt)])
    plsc.subcore_barrier()

    gvs = (gv0, gv1)
    semz = (sz0, sz1)
    semg = (sg0, sg1)

    def fetch(g, b):
        pltpu.sync_copy(si_hbm.at[g], sidx.at[b])
        pltpu.sync_copy(di_hbm.at[g], didx.at[b])
        e0 = pl.multiple_of(g * _CH, _CH)
        return pltpu.async_copy(z_hbm.at[pl.ds(e0, _CH)], gvs[b], semz[b])

    def gather(g, b):
        return pltpu.async_copy(y_hbm.at[sidx.at[b]], gvs[b], semg[b],
                                add=True)

    def consume(g, b):
        _tanh_inplace(gvs[b], d)
        e0 = pl.multiple_of(g * _CH, _CH)
        if write_m:
            pltpu.sync_copy(gvs[b], m_hbm.at[pl.ds(e0, _CH)])
        pltpu.sync_copy(gvs[b], aggr_sh.at[didx.at[b]], add=True)

    base = jnp.where(cid == 0, sid * nch0, _NS * nch0 + sid * nch1)
    pairs = jnp.where(cid == 0, nch0 // 2, nch1 // 2)

    def pair(c, _):
        g0 = base + 2 * c
        g1 = g0 + 1
        f0 = fetch(g0, 0)
        f1 = fetch(g1, 1)
        f0.wait()
        cp0 = gather(g0, 0)
        f1.wait()
        cp1 = gather(g1, 1)
        cp0.wait()
        consume(g0, 0)
        cp1.wait()
        consume(g1, 1)
        return 0

    lax.fori_loop(0, pairs, pair, 0, unroll=False)

    plsc.subcore_barrier()
    pltpu.sync_copy(aggr_sh.at[pl.ds(r0, rpt)],
                    part_hbm.at[cid, pl.ds(r0, rpt)])


def _make_sc_layer(write_m, n_pad, e_pad, d):
    g_tot = e_pad // _CH
    nch0 = 2 * int(round(_FRAC0 * g_tot / _NS / 2))
    nch1 = (g_tot - _NS * nch0) // _NS
    mesh = plsc.VectorSubcoreMesh(core_axis_name="c", subcore_axis_name="s",
                                  num_cores=_NC, num_subcores=_NS)
    part_t = jax.ShapeDtypeStruct((_NC, n_pad, d), jnp.float32)
    if write_m:
        out_type = (jax.ShapeDtypeStruct((e_pad, d), jnp.float32), part_t)
    else:
        out_type = part_t
    scratch = [
        pltpu.VMEM_SHARED((n_pad, d), jnp.float32),
        pltpu.VMEM((2, _CH), jnp.int32),
        pltpu.VMEM((2, _CH), jnp.int32),
        pltpu.VMEM((_CH, d), jnp.float32),
        pltpu.VMEM((_CH, d), jnp.float32),
        pltpu.SemaphoreType.DMA,
        pltpu.SemaphoreType.DMA,
        pltpu.SemaphoreType.DMA,
        pltpu.SemaphoreType.DMA,
        pltpu.SemaphoreType.DMA,
        pltpu.SemaphoreType.DMA,
    ]

    if write_m:
        def body(y, z, si, di, zero, m, part, *s):
            _sc_edge_body(True, n_pad, nch0, nch1, d, y, z, si, di, zero, m,
                          part, *s)
    else:
        def body(y, z, si, di, zero, part, *s):
            _sc_edge_body(False, n_pad, nch0, nch1, d, y, z, si, di, zero,
                          None, part, *s)

    return pl.kernel(body, out_type=out_type, mesh=mesh,
                     scratch_types=scratch)


def kernel(node_feature, edge_index, edge_feature, batch,
           W_emb, b_emb,
           W_msg0, b_msg0, W_upd0, b_upd0,
           W_msg1, b_msg1, W_upd1, b_upd1,
           W_d1, b_d1, W_d2, b_d2):
    n, d = node_feature.shape
    e = edge_index.shape[1]
    fe = edge_feature.shape[1]

    e_pad = _cdiv(e, _NW * _CH * 2) * (_NW * _CH * 2)
    n_pad = _cdiv(n + 1, _NS * 8) * (_NS * 8)
    g_tot = e_pad // _CH

    src = edge_index[0].astype(jnp.int32)
    dst = edge_index[1].astype(jnp.int32)
    src_p = jnp.concatenate(
        [src, jnp.zeros((e_pad - e,), jnp.int32)]).reshape(g_tot, _CH)
    dst_p = jnp.concatenate(
        [dst, jnp.full((e_pad - e,), n, jnp.int32)]).reshape(g_tot, _CH)
    ef_p = jnp.zeros((e_pad, fe), jnp.float32).at[:e].set(edge_feature)
    zeros_hbm = jnp.zeros((n_pad, d), jnp.float32)

    be = b_emb.reshape(1, d)
    bm0 = b_msg0.reshape(1, d)
    bm1 = b_msg1.reshape(1, d)
    bu0 = b_upd0.reshape(1, d)
    bu1 = b_upd1.reshape(1, d)
    bd1 = b_d1.reshape(1, -1)
    bd2 = b_d2.reshape(1, -1)

    sc_l0 = _make_sc_layer(True, n_pad, e_pad, d)
    sc_l1 = _make_sc_layer(False, n_pad, e_pad, d)

    y0 = _node_linear(node_feature, W_msg0[:d])
    z0 = _edge_z0(ef_p, W_emb, be, W_msg0[d:], bm0)
    m0, part0 = sc_l0(y0, z0, src_p, dst_p, zeros_hbm)
    x1, y1 = _update(node_feature, part0, W_upd0[:d], W_upd0[d:], bu0,
                     W_msg1[:d])

    z1 = _edge_z1(m0, W_msg1[d:], bm1)
    part1 = sc_l1(y1, z1, src_p, dst_p, zeros_hbm)

    return _final(x1, part1, W_upd1[:d], W_upd1[d:], bu1,
                  W_d1, bd1, W_d2, bd2)
